# baseline (speedup 1.0000x reference)
import functools

import jax
import jax.numpy as jnp
from jax import lax
from jax.experimental import pallas as pl
from jax.experimental.pallas import tpu as pltpu
from jax.experimental.pallas import tpu_sc as plsc

_N = 1_000_000
_C = 100
_NBINS = 15
_R = 8000
_REAL_BLOCKS = _N // _R
_GRID1 = 128
_NP = _GRID1 * _R
_NW = 32
_E = _NP // _NW
_L = 16
_THRESH = tuple(float(b) / _NBINS for b in range(_NBINS))


def _stage1(probs_ref, labels_ref, conf_ref, d_ref):
    i = pl.program_id(0)

    @pl.when(i < _REAL_BLOCKS)
    def _():
        p = probs_ref[...]
        conf = jnp.max(p, axis=1)
        col = lax.broadcasted_iota(jnp.int32, (_R, _C), 1)
        pred = jnp.min(jnp.where(p == conf[:, None], col, _C), axis=1)
        acc = (pred == labels_ref[0, 0]).astype(jnp.float32)
        conf_ref[0, 0] = conf
        d_ref[0, 0] = conf - acc

    @pl.when(i >= _REAL_BLOCKS)
    def _():
        conf_ref[0, 0] = jnp.zeros((_R,), jnp.float32)
        d_ref[0, 0] = jnp.zeros((_R,), jnp.float32)


@functools.cache
def _make_stage2():
    mesh = plsc.VectorSubcoreMesh(
        core_axis_name="c", subcore_axis_name="s", num_cores=2, num_subcores=16
    )

    @functools.partial(
        pl.kernel,
        out_type=jax.ShapeDtypeStruct((_NW, _NBINS, _L), jnp.float32),
        mesh=mesh,
        scratch_types=[
            pltpu.VMEM((_E,), jnp.float32),
            pltpu.VMEM((_E,), jnp.float32),
            pltpu.VMEM((_NBINS, _L), jnp.float32),
        ],
    )
    def _stage2(conf_hbm, d_hbm, out_hbm, conf_v, d_v, acc_v):
        wid = lax.axis_index("s") * 2 + lax.axis_index("c")
        base = wid * _E
        pltpu.sync_copy(conf_hbm.at[pl.ds(base, _E)], conf_v)
        pltpu.sync_copy(d_hbm.at[pl.ds(base, _E)], d_v)

        zero = jnp.zeros((_L,), jnp.float32)

        def body(i, us):
            cv = conf_v[pl.ds(i * _L, _L)]
            dv = d_v[pl.ds(i * _L, _L)]
            return tuple(
                u + jnp.where(cv > t, dv, 0.0) for u, t in zip(us, _THRESH)
            )

        us = lax.fori_loop(0, _E // _L, body, (zero,) * _NBINS)
        for b in range(_NBINS):
            nxt = us[b + 1] if b + 1 < _NBINS else zero
            acc_v[b, :] = us[b] - nxt
        pltpu.sync_copy(acc_v, out_hbm.at[wid])

    return _stage2


def _stage3(part_ref, out_ref):
    x = part_ref[...]
    s = jnp.sum(jnp.sum(x, axis=0), axis=1)
    ece = jnp.sum(jnp.abs(s)) * (1.0 / _N)
    out_ref[...] = jnp.reshape(ece, (1, 1))


def kernel(probs, labels):
    labels3 = labels.astype(jnp.int32).reshape(_REAL_BLOCKS, 1, _R)

    conf3, d3 = pl.pallas_call(
        _stage1,
        grid=(_GRID1,),
        in_specs=[
            pl.BlockSpec((_R, _C), lambda i: (jnp.minimum(i, _REAL_BLOCKS - 1), 0)),
            pl.BlockSpec(
                (1, 1, _R), lambda i: (jnp.minimum(i, _REAL_BLOCKS - 1), 0, 0)
            ),
        ],
        out_specs=[
            pl.BlockSpec((1, 1, _R), lambda i: (i, 0, 0)),
            pl.BlockSpec((1, 1, _R), lambda i: (i, 0, 0)),
        ],
        out_shape=[
            jax.ShapeDtypeStruct((_GRID1, 1, _R), jnp.float32),
            jax.ShapeDtypeStruct((_GRID1, 1, _R), jnp.float32),
        ],
    )(probs, labels3)

    parts = _make_stage2()(conf3.reshape(_NP), d3.reshape(_NP))

    ece = pl.pallas_call(
        _stage3,
        out_shape=jax.ShapeDtypeStruct((1, 1), jnp.float32),
    )(parts)
    return ece.reshape(1)


# speedup vs baseline: 7.5422x; 7.5422x over previous
import functools

import jax
import jax.numpy as jnp
from jax import lax
from jax.experimental import pallas as pl
from jax.experimental.pallas import tpu as pltpu
from jax.experimental.pallas import tpu_sc as plsc

_N = 1_000_000
_C = 100
_NBINS = 15
_BK = 8192
_GRID1 = -(-_N // _BK)
_L = 16
_NW = 32
_E = 31264
_NP = _NW * _E
_THRESH = tuple(float(b) / _NBINS for b in range(_NBINS))


def _stage1(pt_ref, labels_ref, conf_ref, d_ref):
    p = pt_ref[...]
    conf = jnp.max(p, axis=0)
    row = lax.broadcasted_iota(jnp.int32, (_C, _BK), 0)
    pred = jnp.min(jnp.where(p == conf[None, :], row, _C), axis=0)
    acc = (pred == labels_ref[...]).astype(jnp.float32)
    conf_ref[...] = conf
    d_ref[...] = conf - acc


@functools.cache
def _make_stage2():
    mesh = plsc.VectorSubcoreMesh(
        core_axis_name="c", subcore_axis_name="s", num_cores=2, num_subcores=16
    )

    @functools.partial(
        pl.kernel,
        out_type=jax.ShapeDtypeStruct((_NW, _NBINS, _L), jnp.float32),
        mesh=mesh,
        scratch_types=[
            pltpu.VMEM((_E,), jnp.float32),
            pltpu.VMEM((_E,), jnp.float32),
            pltpu.VMEM((_NBINS, _L), jnp.float32),
        ],
    )
    def _stage2(conf_hbm, d_hbm, out_hbm, conf_v, d_v, acc_v):
        wid = lax.axis_index("s") * 2 + lax.axis_index("c")
        base = wid * _E
        pltpu.sync_copy(conf_hbm.at[pl.ds(base, _E)], conf_v)
        pltpu.sync_copy(d_hbm.at[pl.ds(base, _E)], d_v)

        zero = jnp.zeros((_L,), jnp.float32)

        def body(i, us):
            cv = conf_v[pl.ds(i * _L, _L)]
            dv = d_v[pl.ds(i * _L, _L)]
            return tuple(
                u + jnp.where(cv > t, dv, 0.0) for u, t in zip(us, _THRESH)
            )

        us = lax.fori_loop(0, _E // _L, body, (zero,) * _NBINS)
        for b in range(_NBINS):
            nxt = us[b + 1] if b + 1 < _NBINS else zero
            acc_v[b, :] = us[b] - nxt
        pltpu.sync_copy(acc_v, out_hbm.at[wid])

    return _stage2


def _stage3(part_ref, out_ref):
    x = part_ref[...]
    s = jnp.sum(jnp.sum(x, axis=0), axis=1)
    ece = jnp.sum(jnp.abs(s)) * (1.0 / _N)
    out_ref[...] = jnp.reshape(ece, (1, 1))


def kernel(probs, labels):
    pt = probs.T
    labels1 = labels.astype(jnp.int32)

    conf, d = pl.pallas_call(
        _stage1,
        grid=(_GRID1,),
        in_specs=[
            pl.BlockSpec((_C, _BK), lambda i: (0, i)),
            pl.BlockSpec((_BK,), lambda i: (i,)),
        ],
        out_specs=[
            pl.BlockSpec((_BK,), lambda i: (i,)),
            pl.BlockSpec((_BK,), lambda i: (i,)),
        ],
        out_shape=[
            jax.ShapeDtypeStruct((_N,), jnp.float32),
            jax.ShapeDtypeStruct((_N,), jnp.float32),
        ],
    )(pt, labels1)

    pad = jnp.zeros((_NP - _N,), jnp.float32)
    parts = _make_stage2()(
        jnp.concatenate([conf, pad]), jnp.concatenate([d, pad])
    )

    ece = pl.pallas_call(
        _stage3,
        out_shape=jax.ShapeDtypeStruct((1, 1), jnp.float32),
    )(parts)
    return ece.reshape(1)


# speedup vs baseline: 7.6840x; 1.0188x over previous
import functools

import jax
import jax.numpy as jnp
from jax import lax
from jax.experimental import pallas as pl
from jax.experimental.pallas import tpu as pltpu
from jax.experimental.pallas import tpu_sc as plsc

_N = 1_000_000
_C = 100
_NBINS = 15
_BK = 8192
_GRID1 = -(-_N // _BK)
_L = 16
_NW = 32
_E = 31264
_NP = _NW * _E
_THRESH = tuple(float(b) / _NBINS for b in range(_NBINS))


def _stage1(pt_ref, labels_ref, conf_ref, d_ref):
    p = pt_ref[...]
    conf = jnp.max(p, axis=0)
    row = lax.broadcasted_iota(jnp.int32, (_C, _BK), 0)
    pred = jnp.min(jnp.where(p == conf[None, :], row, _C), axis=0)
    acc = (pred == labels_ref[...]).astype(jnp.float32)
    gidx = pl.program_id(0) * _BK + lax.broadcasted_iota(jnp.int32, (_BK,), 0)
    valid = gidx < _N
    conf_ref[...] = jnp.where(valid, conf, 0.0)
    d_ref[...] = jnp.where(valid, conf - acc, 0.0)


@functools.cache
def _make_stage2():
    mesh = plsc.VectorSubcoreMesh(
        core_axis_name="c", subcore_axis_name="s", num_cores=2, num_subcores=16
    )

    @functools.partial(
        pl.kernel,
        out_type=jax.ShapeDtypeStruct((_NW, _NBINS, _L), jnp.float32),
        mesh=mesh,
        scratch_types=[
            pltpu.VMEM((_E,), jnp.float32),
            pltpu.VMEM((_E,), jnp.float32),
            pltpu.VMEM((_NBINS, _L), jnp.float32),
        ],
    )
    def _stage2(conf_hbm, d_hbm, out_hbm, conf_v, d_v, acc_v):
        wid = lax.axis_index("s") * 2 + lax.axis_index("c")
        base = wid * _E
        pltpu.sync_copy(conf_hbm.at[pl.ds(base, _E)], conf_v)
        pltpu.sync_copy(d_hbm.at[pl.ds(base, _E)], d_v)

        zero = jnp.zeros((_L,), jnp.float32)

        def body(i, us):
            cv = conf_v[pl.ds(i * _L, _L)]
            dv = d_v[pl.ds(i * _L, _L)]
            return tuple(
                u + jnp.where(cv > t, dv, 0.0) for u, t in zip(us, _THRESH)
            )

        us = lax.fori_loop(0, _E // _L, body, (zero,) * _NBINS)
        for b in range(_NBINS):
            nxt = us[b + 1] if b + 1 < _NBINS else zero
            acc_v[b, :] = us[b] - nxt
        pltpu.sync_copy(acc_v, out_hbm.at[wid])

    return _stage2


def _stage3(part_ref, out_ref):
    x = part_ref[...]
    s = jnp.sum(jnp.sum(x, axis=0), axis=1)
    ece = jnp.sum(jnp.abs(s)) * (1.0 / _N)
    out_ref[...] = jnp.reshape(ece, (1, 1))


def kernel(probs, labels):
    pt = probs.T
    labels1 = labels.astype(jnp.int32)

    conf, d = pl.pallas_call(
        _stage1,
        grid=(_GRID1,),
        in_specs=[
            pl.BlockSpec((_C, _BK), lambda i: (0, i)),
            pl.BlockSpec((_BK,), lambda i: (i,)),
        ],
        out_specs=[
            pl.BlockSpec((_BK,), lambda i: (i,)),
            pl.BlockSpec((_BK,), lambda i: (i,)),
        ],
        out_shape=[
            jax.ShapeDtypeStruct((_NP,), jnp.float32),
            jax.ShapeDtypeStruct((_NP,), jnp.float32),
        ],
    )(pt, labels1)

    parts = _make_stage2()(conf, d)

    ece = pl.pallas_call(
        _stage3,
        out_shape=jax.ShapeDtypeStruct((1, 1), jnp.float32),
    )(parts)
    return ece.reshape(1)


# speedup vs baseline: 8.9901x; 1.1700x over previous
import functools

import jax
import jax.numpy as jnp
from jax import lax
from jax.experimental import pallas as pl
from jax.experimental.pallas import tpu as pltpu
from jax.experimental.pallas import tpu_sc as plsc

_N = 1_000_000
_C = 100
_NBINS = 15
_BK = 16384
_GRID1 = -(-_N // _BK)
_L = 16
_NW = 32
_E = 31264
_NP = _NW * _E
_THRESH = tuple(float(b) / _NBINS for b in range(_NBINS))


def _stage1(pt_ref, labels_ref, conf_ref, d_ref):
    p = pt_ref[...]
    conf = jnp.max(p, axis=0)
    row = lax.broadcasted_iota(jnp.int32, (_C, _BK), 0)
    pred = jnp.min(jnp.where(p == conf[None, :], row, _C), axis=0)
    acc = (pred == labels_ref[...]).astype(jnp.float32)
    gidx = pl.program_id(0) * _BK + lax.broadcasted_iota(jnp.int32, (_BK,), 0)
    valid = gidx < _N
    conf_ref[...] = jnp.where(valid, conf, 0.0)
    d_ref[...] = jnp.where(valid, conf - acc, 0.0)


@functools.cache
def _make_stage2():
    mesh = plsc.VectorSubcoreMesh(
        core_axis_name="c", subcore_axis_name="s", num_cores=2, num_subcores=16
    )

    @functools.partial(
        pl.kernel,
        out_type=jax.ShapeDtypeStruct((_NW, _NBINS, _L), jnp.float32),
        mesh=mesh,
        scratch_types=[
            pltpu.VMEM((_E,), jnp.float32),
            pltpu.VMEM((_E,), jnp.float32),
            pltpu.VMEM((_NBINS, _L), jnp.float32),
        ],
    )
    def _stage2(conf_hbm, d_hbm, out_hbm, conf_v, d_v, acc_v):
        wid = lax.axis_index("s") * 2 + lax.axis_index("c")
        base = wid * _E
        pltpu.sync_copy(conf_hbm.at[pl.ds(base, _E)], conf_v)
        pltpu.sync_copy(d_hbm.at[pl.ds(base, _E)], d_v)

        zero = jnp.zeros((_L,), jnp.float32)

        def body(i, us):
            cv = conf_v[pl.ds(i * _L, _L)]
            dv = d_v[pl.ds(i * _L, _L)]
            return tuple(
                u + jnp.where(cv > t, dv, 0.0) for u, t in zip(us, _THRESH)
            )

        us = lax.fori_loop(0, _E // _L, body, (zero,) * _NBINS)
        for b in range(_NBINS):
            nxt = us[b + 1] if b + 1 < _NBINS else zero
            acc_v[b, :] = us[b] - nxt
        pltpu.sync_copy(acc_v, out_hbm.at[wid])

    return _stage2


def _stage3(part_ref, out_ref):
    x = part_ref[...]
    s = jnp.sum(jnp.sum(x, axis=0), axis=1)
    ece = jnp.sum(jnp.abs(s)) * (1.0 / _N)
    out_ref[...] = jnp.reshape(ece, (1, 1))


def kernel(probs, labels):
    pt = probs.T
    labels1 = labels.astype(jnp.int32)

    conf, d = pl.pallas_call(
        _stage1,
        grid=(_GRID1,),
        in_specs=[
            pl.BlockSpec((_C, _BK), lambda i: (0, i)),
            pl.BlockSpec((_BK,), lambda i: (i,)),
        ],
        out_specs=[
            pl.BlockSpec((_BK,), lambda i: (i,)),
            pl.BlockSpec((_BK,), lambda i: (i,)),
        ],
        out_shape=[
            jax.ShapeDtypeStruct((_NP,), jnp.float32),
            jax.ShapeDtypeStruct((_NP,), jnp.float32),
        ],
    )(pt, labels1)

    parts = _make_stage2()(conf, d)

    ece = pl.pallas_call(
        _stage3,
        out_shape=jax.ShapeDtypeStruct((1, 1), jnp.float32),
    )(parts)
    return ece.reshape(1)


# speedup vs baseline: 9.7839x; 1.0883x over previous
import functools

import jax
import jax.numpy as jnp
from jax import lax
from jax.experimental import pallas as pl
from jax.experimental.pallas import tpu as pltpu
from jax.experimental.pallas import tpu_sc as plsc

_N = 1_000_000
_C = 100
_NBINS = 15
_BK = 32768
_GRID1 = -(-_N // _BK)
_L = 16
_NW = 32
_E = 31264
_NP = _NW * _E
_THRESH = tuple(float(b) / _NBINS for b in range(_NBINS))


def _stage1(pt_ref, labels_ref, conf_ref, d_ref):
    p = pt_ref[...]
    conf = jnp.max(p, axis=0)
    row = lax.broadcasted_iota(jnp.int32, (_C, _BK), 0)
    pred = jnp.min(jnp.where(p == conf[None, :], row, _C), axis=0)
    acc = (pred == labels_ref[...]).astype(jnp.float32)
    gidx = pl.program_id(0) * _BK + lax.broadcasted_iota(jnp.int32, (_BK,), 0)
    valid = gidx < _N
    conf_ref[...] = jnp.where(valid, conf, 0.0)
    d_ref[...] = jnp.where(valid, conf - acc, 0.0)


@functools.cache
def _make_stage2():
    mesh = plsc.VectorSubcoreMesh(
        core_axis_name="c", subcore_axis_name="s", num_cores=2, num_subcores=16
    )

    @functools.partial(
        pl.kernel,
        out_type=jax.ShapeDtypeStruct((_NW, _NBINS, _L), jnp.float32),
        mesh=mesh,
        scratch_types=[
            pltpu.VMEM((_E,), jnp.float32),
            pltpu.VMEM((_E,), jnp.float32),
            pltpu.VMEM((_NBINS, _L), jnp.float32),
        ],
    )
    def _stage2(conf_hbm, d_hbm, out_hbm, conf_v, d_v, acc_v):
        wid = lax.axis_index("s") * 2 + lax.axis_index("c")
        base = wid * _E
        pltpu.sync_copy(conf_hbm.at[pl.ds(base, _E)], conf_v)
        pltpu.sync_copy(d_hbm.at[pl.ds(base, _E)], d_v)

        zero = jnp.zeros((_L,), jnp.float32)

        def body(i, us):
            cv = conf_v[pl.ds(i * _L, _L)]
            dv = d_v[pl.ds(i * _L, _L)]
            return tuple(
                u + jnp.where(cv > t, dv, 0.0) for u, t in zip(us, _THRESH)
            )

        us = lax.fori_loop(0, _E // _L, body, (zero,) * _NBINS)
        for b in range(_NBINS):
            nxt = us[b + 1] if b + 1 < _NBINS else zero
            acc_v[b, :] = us[b] - nxt
        pltpu.sync_copy(acc_v, out_hbm.at[wid])

    return _stage2


def _stage3(part_ref, out_ref):
    x = part_ref[...]
    s = jnp.sum(jnp.sum(x, axis=0), axis=1)
    ece = jnp.sum(jnp.abs(s)) * (1.0 / _N)
    out_ref[...] = jnp.reshape(ece, (1, 1))


def kernel(probs, labels):
    pt = probs.T
    labels1 = labels.astype(jnp.int32)

    conf, d = pl.pallas_call(
        _stage1,
        grid=(_GRID1,),
        in_specs=[
            pl.BlockSpec((_C, _BK), lambda i: (0, i)),
            pl.BlockSpec((_BK,), lambda i: (i,)),
        ],
        out_specs=[
            pl.BlockSpec((_BK,), lambda i: (i,)),
            pl.BlockSpec((_BK,), lambda i: (i,)),
        ],
        out_shape=[
            jax.ShapeDtypeStruct((_NP,), jnp.float32),
            jax.ShapeDtypeStruct((_NP,), jnp.float32),
        ],
    )(pt, labels1)

    parts = _make_stage2()(conf, d)

    ece = pl.pallas_call(
        _stage3,
        out_shape=jax.ShapeDtypeStruct((1, 1), jnp.float32),
    )(parts)
    return ece.reshape(1)


# speedup vs baseline: 9.8214x; 1.0038x over previous
import functools

import jax
import jax.numpy as jnp
from jax import lax
from jax.experimental import pallas as pl
from jax.experimental.pallas import tpu as pltpu
from jax.experimental.pallas import tpu_sc as plsc

_N = 1_000_000
_C = 100
_NBINS = 15
_BK = 65536
_GRID1 = -(-_N // _BK)
_L = 16
_NW = 32
_E = 31264
_NP = _NW * _E
_THRESH = tuple(float(b) / _NBINS for b in range(_NBINS))


def _stage1(pt_ref, labels_ref, conf_ref, d_ref):
    p = pt_ref[...]
    conf = jnp.max(p, axis=0)
    row = lax.broadcasted_iota(jnp.int32, (_C, _BK), 0)
    pred = jnp.min(jnp.where(p == conf[None, :], row, _C), axis=0)
    acc = (pred == labels_ref[...]).astype(jnp.float32)
    gidx = pl.program_id(0) * _BK + lax.broadcasted_iota(jnp.int32, (_BK,), 0)
    valid = gidx < _N
    conf_ref[...] = jnp.where(valid, conf, 0.0)
    d_ref[...] = jnp.where(valid, conf - acc, 0.0)


@functools.cache
def _make_stage2():
    mesh = plsc.VectorSubcoreMesh(
        core_axis_name="c", subcore_axis_name="s", num_cores=2, num_subcores=16
    )

    @functools.partial(
        pl.kernel,
        out_type=jax.ShapeDtypeStruct((_NW, _NBINS, _L), jnp.float32),
        mesh=mesh,
        scratch_types=[
            pltpu.VMEM((_E,), jnp.float32),
            pltpu.VMEM((_E,), jnp.float32),
            pltpu.VMEM((_NBINS, _L), jnp.float32),
        ],
    )
    def _stage2(conf_hbm, d_hbm, out_hbm, conf_v, d_v, acc_v):
        wid = lax.axis_index("s") * 2 + lax.axis_index("c")
        base = wid * _E
        pltpu.sync_copy(conf_hbm.at[pl.ds(base, _E)], conf_v)
        pltpu.sync_copy(d_hbm.at[pl.ds(base, _E)], d_v)

        zero = jnp.zeros((_L,), jnp.float32)

        def body(i, us):
            cv = conf_v[pl.ds(i * _L, _L)]
            dv = d_v[pl.ds(i * _L, _L)]
            return tuple(
                u + jnp.where(cv > t, dv, 0.0) for u, t in zip(us, _THRESH)
            )

        us = lax.fori_loop(0, _E // _L, body, (zero,) * _NBINS)
        for b in range(_NBINS):
            nxt = us[b + 1] if b + 1 < _NBINS else zero
            acc_v[b, :] = us[b] - nxt
        pltpu.sync_copy(acc_v, out_hbm.at[wid])

    return _stage2


def _stage3(part_ref, out_ref):
    x = part_ref[...]
    s = jnp.sum(jnp.sum(x, axis=0), axis=1)
    ece = jnp.sum(jnp.abs(s)) * (1.0 / _N)
    out_ref[...] = jnp.reshape(ece, (1, 1))


def kernel(probs, labels):
    pt = probs.T
    labels1 = labels.astype(jnp.int32)

    conf, d = pl.pallas_call(
        _stage1,
        grid=(_GRID1,),
        in_specs=[
            pl.BlockSpec((_C, _BK), lambda i: (0, i)),
            pl.BlockSpec((_BK,), lambda i: (i,)),
        ],
        out_specs=[
            pl.BlockSpec((_BK,), lambda i: (i,)),
            pl.BlockSpec((_BK,), lambda i: (i,)),
        ],
        out_shape=[
            jax.ShapeDtypeStruct((_NP,), jnp.float32),
            jax.ShapeDtypeStruct((_NP,), jnp.float32),
        ],
    )(pt, labels1)

    parts = _make_stage2()(conf, d)

    ece = pl.pallas_call(
        _stage3,
        out_shape=jax.ShapeDtypeStruct((1, 1), jnp.float32),
    )(parts)
    return ece.reshape(1)


# speedup vs baseline: 9.9827x; 1.0164x over previous
import functools

import jax
import jax.numpy as jnp
from jax import lax
from jax.experimental import pallas as pl
from jax.experimental.pallas import tpu as pltpu
from jax.experimental.pallas import tpu_sc as plsc

_N = 1_000_000
_C = 100
_NBINS = 15
_BK = 65536
_L = 16
_NW = 32
_THRESH = tuple(float(b) / _NBINS for b in range(_NBINS))

_SPLIT_BLOCKS = 13
_N1 = _SPLIT_BLOCKS * _BK
_NV2 = _N - _N1
_NP2 = -(-_NV2 // 512) * 512
_CHUNKS = (
    (0, _SPLIT_BLOCKS, _N1, _N1),
    (_SPLIT_BLOCKS, -(-_NV2 // _BK), _NV2, _NP2),
)


def _make_stage1(off_blocks, n_valid):
    def _stage1(pt_ref, labels_ref, conf_ref, d_ref):
        p = pt_ref[...]
        conf = jnp.max(p, axis=0)
        row = lax.broadcasted_iota(jnp.int32, (_C, _BK), 0)
        pred = jnp.min(jnp.where(p == conf[None, :], row, _C), axis=0)
        acc = (pred == labels_ref[...]).astype(jnp.float32)
        gidx = pl.program_id(0) * _BK + lax.broadcasted_iota(
            jnp.int32, (_BK,), 0
        )
        valid = gidx < n_valid
        conf_ref[...] = jnp.where(valid, conf, 0.0)
        d_ref[...] = jnp.where(valid, conf - acc, 0.0)

    return _stage1


def _run_stage1(pt, labels1, off_blocks, grid, n_valid, n_pad):
    return pl.pallas_call(
        _make_stage1(off_blocks, n_valid),
        grid=(grid,),
        in_specs=[
            pl.BlockSpec((_C, _BK), lambda i: (0, i + off_blocks)),
            pl.BlockSpec((_BK,), lambda i: (i + off_blocks,)),
        ],
        out_specs=[
            pl.BlockSpec((_BK,), lambda i: (i,)),
            pl.BlockSpec((_BK,), lambda i: (i,)),
        ],
        out_shape=[
            jax.ShapeDtypeStruct((n_pad,), jnp.float32),
            jax.ShapeDtypeStruct((n_pad,), jnp.float32),
        ],
    )(pt, labels1)


@functools.cache
def _make_stage2(e):
    mesh = plsc.VectorSubcoreMesh(
        core_axis_name="c", subcore_axis_name="s", num_cores=2, num_subcores=16
    )

    @functools.partial(
        pl.kernel,
        out_type=jax.ShapeDtypeStruct((_NW, _NBINS, _L), jnp.float32),
        mesh=mesh,
        scratch_types=[
            pltpu.VMEM((e,), jnp.float32),
            pltpu.VMEM((e,), jnp.float32),
            pltpu.VMEM((_NBINS, _L), jnp.float32),
        ],
    )
    def _stage2(conf_hbm, d_hbm, out_hbm, conf_v, d_v, acc_v):
        wid = lax.axis_index("s") * 2 + lax.axis_index("c")
        base = wid * e
        pltpu.sync_copy(conf_hbm.at[pl.ds(base, e)], conf_v)
        pltpu.sync_copy(d_hbm.at[pl.ds(base, e)], d_v)

        zero = jnp.zeros((_L,), jnp.float32)

        def body(i, us):
            cv = conf_v[pl.ds(i * _L, _L)]
            dv = d_v[pl.ds(i * _L, _L)]
            return tuple(
                u + jnp.where(cv > t, dv, 0.0) for u, t in zip(us, _THRESH)
            )

        us = lax.fori_loop(0, e // _L, body, (zero,) * _NBINS)
        for b in range(_NBINS):
            nxt = us[b + 1] if b + 1 < _NBINS else zero
            acc_v[b, :] = us[b] - nxt
        pltpu.sync_copy(acc_v, out_hbm.at[wid])

    return _stage2


def _stage3(part_ref, out_ref):
    x = part_ref[...]
    s = jnp.sum(jnp.sum(x, axis=0), axis=1)
    ece = jnp.sum(jnp.abs(s)) * (1.0 / _N)
    out_ref[...] = jnp.reshape(ece, (1, 1))


def kernel(probs, labels):
    pt = probs.T
    labels1 = labels.astype(jnp.int32)

    parts = []
    for off, grid, n_valid, n_pad in _CHUNKS:
        conf, d = _run_stage1(pt, labels1, off, grid, n_valid, n_pad)
        parts.append(_make_stage2(n_pad // _NW)(conf, d))

    ece = pl.pallas_call(
        _stage3,
        out_shape=jax.ShapeDtypeStruct((1, 1), jnp.float32),
    )(jnp.concatenate(parts, axis=0))
    return ece.reshape(1)


# speedup vs baseline: 10.1041x; 1.0122x over previous
import functools

import jax
import jax.numpy as jnp
from jax import lax
from jax.experimental import pallas as pl
from jax.experimental.pallas import tpu as pltpu
from jax.experimental.pallas import tpu_sc as plsc

_N = 1_000_000
_C = 100
_NBINS = 15
_BK = 65536
_L = 16
_NW = 32
_THRESH = tuple(float(b) / _NBINS for b in range(_NBINS))

_SPLIT_BLOCKS = 14
_N1 = _SPLIT_BLOCKS * _BK
_NV2 = _N - _N1
_NP2 = -(-_NV2 // 512) * 512
_CHUNKS = (
    (0, _SPLIT_BLOCKS, _N1, _N1),
    (_SPLIT_BLOCKS, -(-_NV2 // _BK), _NV2, _NP2),
)


def _make_stage1(off_blocks, n_valid):
    def _stage1(pt_ref, labels_ref, conf_ref, d_ref):
        p = pt_ref[...]
        conf = jnp.max(p, axis=0)
        row = lax.broadcasted_iota(jnp.int32, (_C, _BK), 0)
        pred = jnp.min(jnp.where(p == conf[None, :], row, _C), axis=0)
        acc = (pred == labels_ref[...]).astype(jnp.float32)
        gidx = pl.program_id(0) * _BK + lax.broadcasted_iota(
            jnp.int32, (_BK,), 0
        )
        valid = gidx < n_valid
        conf_ref[...] = jnp.where(valid, conf, 0.0)
        d_ref[...] = jnp.where(valid, conf - acc, 0.0)

    return _stage1


def _run_stage1(pt, labels1, off_blocks, grid, n_valid, n_pad):
    return pl.pallas_call(
        _make_stage1(off_blocks, n_valid),
        grid=(grid,),
        in_specs=[
            pl.BlockSpec((_C, _BK), lambda i: (0, i + off_blocks)),
            pl.BlockSpec((_BK,), lambda i: (i + off_blocks,)),
        ],
        out_specs=[
            pl.BlockSpec((_BK,), lambda i: (i,)),
            pl.BlockSpec((_BK,), lambda i: (i,)),
        ],
        out_shape=[
            jax.ShapeDtypeStruct((n_pad,), jnp.float32),
            jax.ShapeDtypeStruct((n_pad,), jnp.float32),
        ],
    )(pt, labels1)


@functools.cache
def _make_stage2(e):
    mesh = plsc.VectorSubcoreMesh(
        core_axis_name="c", subcore_axis_name="s", num_cores=2, num_subcores=16
    )

    @functools.partial(
        pl.kernel,
        out_type=jax.ShapeDtypeStruct((_NW, _NBINS, _L), jnp.float32),
        mesh=mesh,
        scratch_types=[
            pltpu.VMEM((e,), jnp.float32),
            pltpu.VMEM((e,), jnp.float32),
            pltpu.VMEM((_NBINS, _L), jnp.float32),
        ],
    )
    def _stage2(conf_hbm, d_hbm, out_hbm, conf_v, d_v, acc_v):
        wid = lax.axis_index("s") * 2 + lax.axis_index("c")
        base = wid * e
        pltpu.sync_copy(conf_hbm.at[pl.ds(base, e)], conf_v)
        pltpu.sync_copy(d_hbm.at[pl.ds(base, e)], d_v)

        zero = jnp.zeros((_L,), jnp.float32)

        def body(i, us):
            cv = conf_v[pl.ds(i * _L, _L)]
            dv = d_v[pl.ds(i * _L, _L)]
            return tuple(
                u + jnp.where(cv > t, dv, 0.0) for u, t in zip(us, _THRESH)
            )

        us = lax.fori_loop(0, e // _L, body, (zero,) * _NBINS)
        for b in range(_NBINS):
            nxt = us[b + 1] if b + 1 < _NBINS else zero
            acc_v[b, :] = us[b] - nxt
        pltpu.sync_copy(acc_v, out_hbm.at[wid])

    return _stage2


def _stage3(pa_ref, pb_ref, out_ref):
    x = pa_ref[...] + pb_ref[...]
    s = jnp.sum(jnp.sum(x, axis=0), axis=1)
    ece = jnp.sum(jnp.abs(s)) * (1.0 / _N)
    out_ref[...] = jnp.reshape(ece, (1, 1))


def kernel(probs, labels):
    pt = probs.T
    labels1 = labels.astype(jnp.int32)

    parts = []
    for off, grid, n_valid, n_pad in _CHUNKS:
        conf, d = _run_stage1(pt, labels1, off, grid, n_valid, n_pad)
        parts.append(_make_stage2(n_pad // _NW)(conf, d))

    ece = pl.pallas_call(
        _stage3,
        out_shape=jax.ShapeDtypeStruct((1, 1), jnp.float32),
    )(*parts)
    return ece.reshape(1)
